# baseline (speedup 1.0000x reference)
import functools

import jax
import jax.numpy as jnp
from jax import lax
from jax.experimental import pallas as pl
from jax.experimental.pallas import tpu as pltpu
from jax.experimental.pallas import tpu_sc as plsc

N = 10000
NP = 10240
D = 128
E = 320000
EP = 327680

NSUB = 16
EPS = EP // NSUB
CH = 128
CHH = CH // 2
NCHUNK = EPS // CH
RPS = NP // NSUB
NSEG = 4
SEGCH = NCHUNK // NSEG
SEGE = SEGCH * CH

_f32 = jnp.float32
_i32 = jnp.int32

_mesh = plsc.VectorSubcoreMesh(core_axis_name="c", subcore_axis_name="s")


def _stage_idx(all_ref, base, buf, n):
    for k in range(n // 16):
        buf[pl.ds(16 * k, 16)] = all_ref[pl.ds(base + 16 * k, 16)]


def _agg_body(x_hbm, src0, dst0, src1, dst1, z_hbm, out0, out1,
              sidx_all, didx_all,
              sa0, sb0, da0, db0, sa1, sb1, da1, db1,
              rowsa0, rowsb0, rowsa1, rowsb1,
              accum, sema0, semb0, sema1, semb1):
    c = lax.axis_index("c")
    s = lax.axis_index("s")

    pltpu.sync_copy(z_hbm, accum.at[pl.ds(s * RPS, RPS)])

    bufs0 = (sa0, sb0, da0, db0, rowsa0, rowsb0, sema0, semb0)
    bufs1 = (sa1, sb1, da1, db1, rowsa1, rowsb1, sema1, semb1)

    def do_rel(src, dst):
        plsc.subcore_barrier()

        def fire(g, bufs):
            sa, sb, da, db, rowsa, rowsb, sema, semb = bufs
            _stage_idx(sidx_all, g * CH, sa, CHH)
            _stage_idx(sidx_all, g * CH + CHH, sb, CHH)
            _stage_idx(didx_all, g * CH, da, CHH)
            _stage_idx(didx_all, g * CH + CHH, db, CHH)
            pltpu.async_copy(x_hbm.at[sa], rowsa, sema)
            pltpu.async_copy(x_hbm.at[sb], rowsb, semb)

        def drain(bufs):
            sa, sb, da, db, rowsa, rowsb, sema, semb = bufs
            pltpu.make_async_copy(x_hbm.at[sa], rowsa, sema).wait()
            pltpu.sync_copy(rowsa, accum.at[da], add=True)
            pltpu.make_async_copy(x_hbm.at[sb], rowsb, semb).wait()
            pltpu.sync_copy(rowsb, accum.at[db], add=True)

        def seg_body(q, carry):
            base = s * EPS + q * SEGE
            pltpu.sync_copy(src.at[pl.ds(base, SEGE)], sidx_all)
            pltpu.sync_copy(dst.at[pl.ds(base, SEGE)], didx_all)
            fire(0, bufs0)

            def pair(t, carry2):
                fire(2 * t + 1, bufs1)
                drain(bufs0)

                @pl.when(t + 1 < SEGCH // 2)
                def _():
                    fire(2 * t + 2, bufs0)
                drain(bufs1)
                return carry2

            lax.fori_loop(0, SEGCH // 2, pair, 0)
            return carry

        lax.fori_loop(0, NSEG, seg_body, 0)

    def rel0():
        do_rel(src0, dst0)

    def rel1():
        do_rel(src1, dst1)

    pl.when(c == 0)(rel0)
    pl.when(c == 1)(rel1)
    plsc.subcore_barrier()

    sl = pl.ds(s * RPS, RPS)

    @pl.when(c == 0)
    def _():
        pltpu.sync_copy(accum.at[sl], out0.at[sl])

    @pl.when(c == 1)
    def _():
        pltpu.sync_copy(accum.at[sl], out1.at[sl])


_agg_call = pl.kernel(
    _agg_body,
    out_type=(
        jax.ShapeDtypeStruct((NP, D), _f32),
        jax.ShapeDtypeStruct((NP, D), _f32),
    ),
    mesh=_mesh,
    scratch_types=(
        [pltpu.VMEM((SEGE,), _i32)] * 2
        + [pltpu.VMEM((CHH,), _i32)] * 8
        + [pltpu.VMEM((CHH, D), _f32)] * 4
        + [pltpu.VMEM_SHARED((NP, D), _f32)]
        + [pltpu.SemaphoreType.DMA] * 4
    ),
)


def _cnt_body(dst0, dst1, z_hbm, ones_hbm, cnt0, cnt1,
              didx_all, didx, ones_v, cntacc):
    c = lax.axis_index("c")
    s = lax.axis_index("s")

    pltpu.sync_copy(z_hbm, cntacc.at[pl.ds(s * RPS, RPS)])
    pltpu.sync_copy(ones_hbm, ones_v)

    def do_rel(dst):
        pltpu.sync_copy(dst.at[pl.ds(s * EPS, EPS)], didx_all)
        plsc.subcore_barrier()

        def chunk(g, carry):
            _stage_idx(didx_all, g * CH, didx, CH)
            pltpu.sync_copy(ones_v, cntacc.at[didx], add=True)
            return carry
        lax.fori_loop(0, NCHUNK, chunk, 0)

    def rel0():
        do_rel(dst0)

    def rel1():
        do_rel(dst1)

    pl.when(c == 0)(rel0)
    pl.when(c == 1)(rel1)
    plsc.subcore_barrier()

    sl = pl.ds(s * RPS, RPS)

    @pl.when(c == 0)
    def _():
        pltpu.sync_copy(cntacc.at[sl], cnt0.at[sl])

    @pl.when(c == 1)
    def _():
        pltpu.sync_copy(cntacc.at[sl], cnt1.at[sl])


_cnt_call = pl.kernel(
    _cnt_body,
    out_type=(
        jax.ShapeDtypeStruct((NP, D), _f32),
        jax.ShapeDtypeStruct((NP, D), _f32),
    ),
    mesh=_mesh,
    scratch_types=[
        pltpu.VMEM((EPS,), _i32),
        pltpu.VMEM((CH,), _i32),
        pltpu.VMEM((CH, D), _f32),
        pltpu.VMEM_SHARED((NP, D), _f32),
    ],
)

R = 1000


def _mm_body(relu, h_ref, a0_ref, a1_ref, c0_ref, c1_ref,
             wl0_ref, wl1_ref, wr_ref, b_ref, o_ref):
    c0 = jnp.maximum(c0_ref[...], 1.0)
    c1 = jnp.maximum(c1_ref[...], 1.0)
    m0 = a0_ref[...] / c0
    m1 = a1_ref[...] / c1
    acc = jnp.dot(m0, wl0_ref[...], preferred_element_type=_f32)
    acc = acc + jnp.dot(m1, wl1_ref[...], preferred_element_type=_f32)
    acc = acc + jnp.dot(h_ref[...], wr_ref[...], preferred_element_type=_f32)
    acc = acc + b_ref[...]
    if relu:
        acc = jnp.maximum(acc, 0.0)
    o_ref[...] = acc


def _make_mm(relu):
    row_spec = pl.BlockSpec((R, D), lambda i: (i, 0))
    cnt_spec = pl.BlockSpec((R, 1), lambda i: (i, 0))
    w_spec = pl.BlockSpec((D, D), lambda i: (0, 0))
    b_spec = pl.BlockSpec((1, D), lambda i: (0, 0))
    return pl.pallas_call(
        functools.partial(_mm_body, relu),
        grid=(N // R,),
        in_specs=[row_spec, row_spec, row_spec, cnt_spec, cnt_spec,
                  w_spec, w_spec, w_spec, b_spec],
        out_specs=row_spec,
        out_shape=jax.ShapeDtypeStruct((N, D), _f32),
    )


_mm_relu = _make_mm(True)
_mm_plain = _make_mm(False)


def kernel(x, edge_index_rel0, edge_index_rel1,
           W_l_0_0, b_l_0_0, W_r_0_0, W_l_0_1, b_l_0_1, W_r_0_1,
           W_l_1_0, b_l_1_0, W_r_1_0, W_l_1_1, b_l_1_1, W_r_1_1):
    zeros = jnp.zeros((RPS, D), _f32)
    ones = jnp.ones((CH, D), _f32)

    pad_src = jnp.zeros((EP - E,), _i32)
    pad_dst = jnp.full((EP - E,), NP - 1, _i32)
    s0 = jnp.concatenate([edge_index_rel0[0], pad_src])
    d0 = jnp.concatenate([edge_index_rel0[1], pad_dst])
    s1 = jnp.concatenate([edge_index_rel1[0], pad_src])
    d1 = jnp.concatenate([edge_index_rel1[1], pad_dst])

    cnt0, cnt1 = _cnt_call(d0, d1, zeros, ones)
    c0 = cnt0[:N, :1]
    c1 = cnt1[:N, :1]

    a0, a1 = _agg_call(x, s0, d0, s1, d1, zeros)
    h1 = _mm_relu(x, a0[:N], a1[:N], c0, c1,
                  W_l_0_0, W_l_0_1, W_r_0_0 + W_r_0_1,
                  (b_l_0_0 + b_l_0_1)[None, :])

    a0, a1 = _agg_call(h1, s0, d0, s1, d1, zeros)
    out = _mm_plain(h1, a0[:N], a1[:N], c0, c1,
                    W_l_1_0, W_l_1_1, W_r_1_0 + W_r_1_1,
                    (b_l_1_0 + b_l_1_1)[None, :])
    return out


# speedup vs baseline: 2.9779x; 2.9779x over previous
import functools

import jax
import jax.numpy as jnp
from jax import lax
from jax.experimental import pallas as pl
from jax.experimental.pallas import tpu as pltpu
from jax.experimental.pallas import tpu_sc as plsc

N = 10000
NP = 10240
D = 128
E = 320000

NSUB = 16
EPS = E // NSUB
CH = 80
NCHUNK = EPS // CH
RPS = NP // NSUB
NSEG = 5
SEGCH = NCHUNK // NSEG
SEGE = SEGCH * CH
NBUF = 3

_f32 = jnp.float32
_i32 = jnp.int32

_mesh = plsc.VectorSubcoreMesh(core_axis_name="c", subcore_axis_name="s")


def _stage_idx(all_ref, base, buf, n):
    for k in range(n // 16):
        buf[pl.ds(16 * k, 16)] = all_ref[pl.ds(base + 16 * k, 16)]


def _agg_body(x_hbm, src0, dst0, src1, dst1, z_hbm, out0, out1,
              sidx_all, didx_all,
              si0, si1, si2, di0, di1, di2,
              rows0, rows1, rows2,
              accum, sem0, sem1, sem2):
    c = lax.axis_index("c")
    s = lax.axis_index("s")

    pltpu.sync_copy(z_hbm, accum.at[pl.ds(s * RPS, RPS)])

    bufs = ((si0, di0, rows0, sem0),
            (si1, di1, rows1, sem1),
            (si2, di2, rows2, sem2))

    def do_rel(src, dst):
        plsc.subcore_barrier()

        def fire(g, b):
            si, di, rows, sem = bufs[b]
            _stage_idx(sidx_all, g * CH, si, CH)
            _stage_idx(didx_all, g * CH, di, CH)
            pltpu.async_copy(x_hbm.at[si], rows, sem)

        def drain(b):
            si, di, rows, sem = bufs[b]
            pltpu.make_async_copy(x_hbm.at[si], rows, sem).wait()
            pltpu.sync_copy(rows, accum.at[di], add=True)

        def seg_body(q, carry):
            base = s * EPS + q * SEGE
            pltpu.sync_copy(src.at[pl.ds(base, SEGE)], sidx_all)
            pltpu.sync_copy(dst.at[pl.ds(base, SEGE)], didx_all)
            fire(0, 0)
            fire(1, 1)

            def triple(u, carry2):
                for j in range(NBUF):
                    g = NBUF * u + j

                    @pl.when(g + 2 < SEGCH)
                    def _(g=g, j=j):
                        fire(g + 2, (j + 2) % NBUF)

                    @pl.when(g < SEGCH)
                    def _(g=g, j=j):
                        drain(j)
                return carry2

            lax.fori_loop(0, (SEGCH + NBUF - 1) // NBUF, triple, 0)
            return carry

        lax.fori_loop(0, NSEG, seg_body, 0)

    def rel0():
        do_rel(src0, dst0)

    def rel1():
        do_rel(src1, dst1)

    pl.when(c == 0)(rel0)
    pl.when(c == 1)(rel1)
    plsc.subcore_barrier()

    sl = pl.ds(s * RPS, RPS)

    @pl.when(c == 0)
    def _():
        pltpu.sync_copy(accum.at[sl], out0.at[sl])

    @pl.when(c == 1)
    def _():
        pltpu.sync_copy(accum.at[sl], out1.at[sl])


_agg_call = pl.kernel(
    _agg_body,
    out_type=(
        jax.ShapeDtypeStruct((NP, D), _f32),
        jax.ShapeDtypeStruct((NP, D), _f32),
    ),
    mesh=_mesh,
    scratch_types=(
        [pltpu.VMEM((SEGE,), _i32)] * 2
        + [pltpu.VMEM((CH,), _i32)] * 6
        + [pltpu.VMEM((CH, D), _f32)] * 3
        + [pltpu.VMEM_SHARED((NP, D), _f32)]
        + [pltpu.SemaphoreType.DMA] * 3
    ),
)


def _cnt_body(dst0, dst1, z_hbm, ones_hbm, cnt0, cnt1,
              didx_all, didx, ones_v, cntacc):
    c = lax.axis_index("c")
    s = lax.axis_index("s")

    pltpu.sync_copy(z_hbm, cntacc.at[pl.ds(s * RPS, RPS)])
    pltpu.sync_copy(ones_hbm, ones_v)

    def do_rel(dst):
        pltpu.sync_copy(dst.at[pl.ds(s * EPS, EPS)], didx_all)
        plsc.subcore_barrier()

        def chunk(g, carry):
            _stage_idx(didx_all, g * CH, didx, CH)
            pltpu.sync_copy(ones_v, cntacc.at[didx], add=True)
            return carry
        lax.fori_loop(0, NCHUNK, chunk, 0)

    def rel0():
        do_rel(dst0)

    def rel1():
        do_rel(dst1)

    pl.when(c == 0)(rel0)
    pl.when(c == 1)(rel1)
    plsc.subcore_barrier()

    sl = pl.ds(s * RPS, RPS)

    @pl.when(c == 0)
    def _():
        pltpu.sync_copy(cntacc.at[sl], cnt0.at[sl])

    @pl.when(c == 1)
    def _():
        pltpu.sync_copy(cntacc.at[sl], cnt1.at[sl])


_cnt_call = pl.kernel(
    _cnt_body,
    out_type=(
        jax.ShapeDtypeStruct((NP, D), _f32),
        jax.ShapeDtypeStruct((NP, D), _f32),
    ),
    mesh=_mesh,
    scratch_types=[
        pltpu.VMEM((EPS,), _i32),
        pltpu.VMEM((CH,), _i32),
        pltpu.VMEM((CH, D), _f32),
        pltpu.VMEM_SHARED((NP, D), _f32),
    ],
)

R = 1000


def _mm_body(relu, h_ref, a0_ref, a1_ref, c0_ref, c1_ref,
             wl0_ref, wl1_ref, wr_ref, b_ref, o_ref):
    c0 = jnp.maximum(c0_ref[...], 1.0)
    c1 = jnp.maximum(c1_ref[...], 1.0)
    m0 = a0_ref[...] / c0
    m1 = a1_ref[...] / c1
    acc = jnp.dot(m0, wl0_ref[...], preferred_element_type=_f32)
    acc = acc + jnp.dot(m1, wl1_ref[...], preferred_element_type=_f32)
    acc = acc + jnp.dot(h_ref[...], wr_ref[...], preferred_element_type=_f32)
    acc = acc + b_ref[...]
    if relu:
        acc = jnp.maximum(acc, 0.0)
    o_ref[...] = acc


def _make_mm(relu):
    row_spec = pl.BlockSpec((R, D), lambda i: (i, 0))
    cnt_spec = pl.BlockSpec((R, 1), lambda i: (i, 0))
    w_spec = pl.BlockSpec((D, D), lambda i: (0, 0))
    b_spec = pl.BlockSpec((1, D), lambda i: (0, 0))
    return pl.pallas_call(
        functools.partial(_mm_body, relu),
        grid=(N // R,),
        in_specs=[row_spec, row_spec, row_spec, cnt_spec, cnt_spec,
                  w_spec, w_spec, w_spec, b_spec],
        out_specs=row_spec,
        out_shape=jax.ShapeDtypeStruct((N, D), _f32),
    )


_mm_relu = _make_mm(True)
_mm_plain = _make_mm(False)


def kernel(x, edge_index_rel0, edge_index_rel1,
           W_l_0_0, b_l_0_0, W_r_0_0, W_l_0_1, b_l_0_1, W_r_0_1,
           W_l_1_0, b_l_1_0, W_r_1_0, W_l_1_1, b_l_1_1, W_r_1_1):
    zeros = jnp.zeros((RPS, D), _f32)
    ones = jnp.ones((CH, D), _f32)

    s0, d0 = edge_index_rel0[0], edge_index_rel0[1]
    s1, d1 = edge_index_rel1[0], edge_index_rel1[1]

    cnt0, cnt1 = _cnt_call(d0, d1, zeros, ones)
    c0 = cnt0[:N, :1]
    c1 = cnt1[:N, :1]

    a0, a1 = _agg_call(x, s0, d0, s1, d1, zeros)
    h1 = _mm_relu(x, a0[:N], a1[:N], c0, c1,
                  W_l_0_0, W_l_0_1, W_r_0_0 + W_r_0_1,
                  (b_l_0_0 + b_l_0_1)[None, :])

    a0, a1 = _agg_call(h1, s0, d0, s1, d1, zeros)
    out = _mm_plain(h1, a0[:N], a1[:N], c0, c1,
                    W_l_1_0, W_l_1_1, W_r_1_0 + W_r_1_1,
                    (b_l_1_0 + b_l_1_1)[None, :])
    return out
